# Initial kernel scaffold; baseline (speedup 1.0000x reference)
import jax
import jax.numpy as jnp
from jax.experimental import pallas as pl


def kernel(x, edge_index, edge_attr, batch, node_W, node_b, Wq, bq, Wk, bk, Wv, bv, We, be, Wskip, bskip, bn_gamma, bn_beta, reg_W1, reg_b1, reg_W2, reg_b2):
    raise NotImplementedError("write your pallas kernel here")


# speedup vs baseline: 21.2449x; 21.2449x over previous
import functools

import jax
import jax.numpy as jnp
from jax import lax
from jax.experimental import pallas as pl
from jax.experimental.pallas import tpu as pltpu
from jax.experimental.pallas import tpu_sc as plsc

N = 10000
E = 320000
DIN = 128
DE = 16
HID = 64
HEADS = 4
CH = 16
NLAYERS = 3
G = 16

NC = 2
NS = 16
NW = NC * NS
EPW = E // NW
CHUNK = 80
NCHUNK = EPW // CHUNK
AW = 128
HALF = N // NC
HROWS = 5120
RPT = HROWS // NS
TRASH = HALF + 16
CPS = E // CHUNK // NS

_mesh = plsc.VectorSubcoreMesh(core_axis_name="c", subcore_axis_name="s")


def _mm(x, w, b, bm):
    m, k = x.shape
    nout = w.shape[1]

    def body(xr, wr, br, outr):
        outr[...] = jnp.dot(xr[...], wr[...],
                            preferred_element_type=jnp.float32, precision=lax.Precision.HIGHEST) + br[...]

    return pl.pallas_call(
        body,
        grid=(m // bm,),
        in_specs=[
            pl.BlockSpec((bm, k), lambda i: (i, 0)),
            pl.BlockSpec((k, nout), lambda i: (0, 0)),
            pl.BlockSpec((1, nout), lambda i: (0, 0)),
        ],
        out_specs=pl.BlockSpec((bm, nout), lambda i: (i, 0)),
        out_shape=jax.ShapeDtypeStruct((m, nout), jnp.float32),
    )(x, w, b.reshape(1, nout))


def _qkv(h, wq, bq, wk, bk, wv, bv):
    bm = 1000

    def body(hr, wqr, bqr, wkr, bkr, wvr, bvr, qr, kvr):
        hh = hr[...]
        qq = jnp.dot(hh, wqr[...], preferred_element_type=jnp.float32, precision=lax.Precision.HIGHEST) + bqr[...]
        qr[...] = jnp.concatenate([qq, jnp.zeros((bm, HID), jnp.float32)], axis=1)
        kk = jnp.dot(hh, wkr[...], preferred_element_type=jnp.float32, precision=lax.Precision.HIGHEST) + bkr[...]
        vv = jnp.dot(hh, wvr[...], preferred_element_type=jnp.float32, precision=lax.Precision.HIGHEST) + bvr[...]
        kvr[...] = jnp.concatenate([kk, vv], axis=1)

    wspec = pl.BlockSpec((HID, HID), lambda i: (0, 0))
    bspec = pl.BlockSpec((1, HID), lambda i: (0, 0))
    return pl.pallas_call(
        body,
        grid=(N // bm,),
        in_specs=[pl.BlockSpec((bm, HID), lambda i: (i, 0)),
                  wspec, bspec, wspec, bspec, wspec, bspec],
        out_specs=[pl.BlockSpec((bm, 2 * HID), lambda i: (i, 0)),
                   pl.BlockSpec((bm, 2 * HID), lambda i: (i, 0))],
        out_shape=[jax.ShapeDtypeStruct((N, 2 * HID), jnp.float32),
                   jax.ShapeDtypeStruct((N, 2 * HID), jnp.float32)],
    )(h, wq, bq.reshape(1, HID), wk, bk.reshape(1, HID), wv, bv.reshape(1, HID))


def _sel():
    row = lax.broadcasted_iota(jnp.int32, (HID, HEADS), 0) // CH
    col = lax.broadcasted_iota(jnp.int32, (HID, HEADS), 1)
    return (row == col).astype(jnp.float32)


def _selT():
    row = lax.broadcasted_iota(jnp.int32, (HEADS, HID), 0)
    col = lax.broadcasted_iota(jnp.int32, (HEADS, HID), 1) // CH
    return (row == col).astype(jnp.float32)


def _edge_math(qd, kvs, e):
    be_blk = 4000

    def body(qdr, kvsr, er, outr):
        ks = kvsr[:, :HID]
        vs = kvsr[:, HID:]
        ee = er[...]
        p = qdr[:, :HID] * (ks + ee)
        alpha = jnp.dot(p, _sel(), preferred_element_type=jnp.float32, precision=lax.Precision.HIGHEST) * 0.25
        ex = jnp.exp(alpha)
        exb = jnp.dot(ex, _selT(), preferred_element_type=jnp.float32, precision=lax.Precision.HIGHEST)
        msg = (vs + ee) * exb
        outr[...] = jnp.concatenate(
            [msg, ex, jnp.zeros((be_blk, AW - HID - HEADS), jnp.float32)], axis=1)

    return pl.pallas_call(
        body,
        grid=(E // be_blk,),
        in_specs=[pl.BlockSpec((be_blk, 2 * HID), lambda i: (i, 0)),
                  pl.BlockSpec((be_blk, 2 * HID), lambda i: (i, 0)),
                  pl.BlockSpec((be_blk, HID), lambda i: (i, 0))],
        out_specs=pl.BlockSpec((be_blk, AW), lambda i: (i, 0)),
        out_shape=jax.ShapeDtypeStruct((E, AW), jnp.float32),
    )(qd, kvs, e)


def _node_update(acc, h, wskip, bskip, gamma, beta):
    bm = 1000
    bpc = HALF // bm

    def body(ar, hr, wr, br, gr, btr, outr):
        a = ar[...].reshape(bm, AW)
        num = a[:, :HID]
        den = a[:, HID:HID + HEADS]
        denb = jnp.dot(den, _selT(), preferred_element_type=jnp.float32, precision=lax.Precision.HIGHEST) + 1e-16
        hh = hr[...]
        out = num / denb + jnp.dot(hh, wr[...],
                                   preferred_element_type=jnp.float32, precision=lax.Precision.HIGHEST) + br[...]
        out = out * gr[...] + btr[...]
        g = out * 0.5 * (1.0 + lax.erf(out * 0.7071067811865476))
        outr[...] = g + hh

    return pl.pallas_call(
        body,
        grid=(N // bm,),
        in_specs=[pl.BlockSpec((1, bm, AW), lambda i: (i // bpc, i % bpc, 0)),
                  pl.BlockSpec((bm, HID), lambda i: (i, 0)),
                  pl.BlockSpec((HID, HID), lambda i: (0, 0)),
                  pl.BlockSpec((1, HID), lambda i: (0, 0)),
                  pl.BlockSpec((1, HID), lambda i: (0, 0)),
                  pl.BlockSpec((1, HID), lambda i: (0, 0))],
        out_specs=pl.BlockSpec((bm, HID), lambda i: (i, 0)),
        out_shape=jax.ShapeDtypeStruct((N, HID), jnp.float32),
    )(acc, h, wskip, bskip.reshape(1, HID),
      gamma.reshape(1, HID), beta.reshape(1, HID))


def _pool_mlp(h, batch3, w1, b1, w2, b2):
    bm = 1000
    steps = N // bm

    def body(hr, br, w1r, b1r, w2r, b2r, outr, sums, counts):
        i = pl.program_id(0)

        @pl.when(i == 0)
        def _init():
            sums[...] = jnp.zeros((G, HID), jnp.float32)
            counts[...] = jnp.zeros((G, 1), jnp.float32)

        bvals = br[...].reshape(1, bm)
        gids = lax.broadcasted_iota(jnp.int32, (G, bm), 0)
        onehot = (jnp.broadcast_to(bvals, (G, bm)) == gids).astype(jnp.float32)
        sums[...] += jnp.dot(onehot, hr[...], preferred_element_type=jnp.float32, precision=lax.Precision.HIGHEST)
        counts[...] += jnp.sum(onehot, axis=1, keepdims=True)

        @pl.when(i == steps - 1)
        def _fin():
            s = sums[...]
            mean = s / jnp.maximum(counts[...], 1.0)
            g = jnp.concatenate([mean, s], axis=1)
            r = jnp.maximum(
                jnp.dot(g, w1r[...], preferred_element_type=jnp.float32, precision=lax.Precision.HIGHEST) + b1r[...],
                0.0)
            outr[...] = jnp.dot(r, w2r[...],
                                preferred_element_type=jnp.float32, precision=lax.Precision.HIGHEST) + b2r[...]

    return pl.pallas_call(
        body,
        grid=(steps,),
        in_specs=[pl.BlockSpec((bm, HID), lambda i: (i, 0)),
                  pl.BlockSpec((1, 1, bm), lambda i: (i, 0, 0)),
                  pl.BlockSpec((2 * HID, HID), lambda i: (0, 0)),
                  pl.BlockSpec((1, HID), lambda i: (0, 0)),
                  pl.BlockSpec((HID, 1), lambda i: (0, 0)),
                  pl.BlockSpec((1, 1), lambda i: (0, 0))],
        out_specs=pl.BlockSpec((G, 1), lambda i: (0, 0)),
        out_shape=jax.ShapeDtypeStruct((G, 1), jnp.float32),
        scratch_shapes=[pltpu.VMEM((G, HID), jnp.float32),
                        pltpu.VMEM((G, 1), jnp.float32)],
    )(h, batch3, w1, b1.reshape(1, HID), w2, b2.reshape(1, 1))


@functools.partial(
    pl.kernel,
    out_type=[jax.ShapeDtypeStruct((E, 2 * HID), jnp.float32),
              jax.ShapeDtypeStruct((E, 2 * HID), jnp.float32)],
    mesh=_mesh,
    scratch_types=[pltpu.VMEM((CHUNK,), jnp.int32),
                   pltpu.VMEM((CHUNK,), jnp.int32),
                   pltpu.VMEM((CHUNK, 2 * HID), jnp.float32),
                   pltpu.VMEM((CHUNK, 2 * HID), jnp.float32),
                   pltpu.SemaphoreType.DMA,
                   pltpu.SemaphoreType.DMA],
)
def _sc_gather(q_hbm, kv_hbm, src3, dst3, qd_out, kvs_out,
               sidx, didx, qbuf, kvbuf, sem1, sem2):
    wid = lax.axis_index("s") * NC + lax.axis_index("c")

    def step(i, carry):
        base = wid * EPW + i * CHUNK
        pltpu.sync_copy(src3.at[wid, i], sidx)
        pltpu.sync_copy(dst3.at[wid, i], didx)
        d1 = pltpu.async_copy(q_hbm.at[didx], qbuf, sem1)
        d2 = pltpu.async_copy(kv_hbm.at[sidx], kvbuf, sem2)
        d1.wait()
        d2.wait()
        pltpu.sync_copy(qbuf, qd_out.at[pl.ds(base, CHUNK)])
        pltpu.sync_copy(kvbuf, kvs_out.at[pl.ds(base, CHUNK)])
        return carry

    lax.fori_loop(0, NCHUNK, step, 0)


@functools.partial(
    pl.kernel,
    out_type=jax.ShapeDtypeStruct((NC, HROWS, AW), jnp.float32),
    mesh=_mesh,
    scratch_types=[pltpu.VMEM((CHUNK,), jnp.int32),
                   pltpu.VMEM((1, CHUNK), jnp.int32),
                   pltpu.VMEM((CHUNK, AW), jnp.float32),
                   pltpu.VMEM((RPT, AW), jnp.float32),
                   pltpu.VMEM_SHARED((HROWS, AW), jnp.float32)],
)
def _sc_scatter(dst2, msgden, out_hbm, idxbuf, lidxbuf, rowbuf, zbuf, shared):
    c = lax.axis_index("c")
    s = lax.axis_index("s")
    lo = c * HALF

    def zrow(i, carry):
        for j in range(AW // 16):
            zbuf[i, pl.ds(j * 16, 16)] = jnp.zeros((16,), jnp.float32)
        return carry

    lax.fori_loop(0, RPT, zrow, 0)
    pltpu.sync_copy(zbuf, shared.at[pl.ds(s * RPT, RPT)])
    plsc.subcore_barrier()

    def step(i, carry):
        ci = s * CPS + i
        pltpu.sync_copy(dst2.at[ci], idxbuf)
        pltpu.sync_copy(msgden.at[pl.ds(ci * CHUNK, CHUNK)], rowbuf)
        for j in range(CHUNK // 16):
            v = idxbuf[pl.ds(j * 16, 16)] - lo
            ok = (v >= 0) & (v < HALF)
            lidxbuf[0, pl.ds(j * 16, 16)] = jnp.where(ok, v, TRASH)
        pltpu.sync_copy(rowbuf, shared.at[lidxbuf.at[0]], add=True)
        return carry

    lax.fori_loop(0, CPS, step, 0)
    plsc.subcore_barrier()
    pltpu.sync_copy(shared.at[pl.ds(s * RPT, RPT)],
                    out_hbm.at[c, pl.ds(s * RPT, RPT)])


def kernel(x, edge_index, edge_attr, batch, node_W, node_b, Wq, bq, Wk, bk,
           Wv, bv, We, be, Wskip, bskip, bn_gamma, bn_beta, reg_W1, reg_b1,
           reg_W2, reg_b2):
    src3 = edge_index[0].reshape(NW, NCHUNK, CHUNK)
    dst3 = edge_index[1].reshape(NW, NCHUNK, CHUNK)
    dst2 = edge_index[1].reshape(E // CHUNK, CHUNK)
    batch3 = batch.reshape(N // 1000, 1, 1000)

    h = _mm(x, node_W, node_b, 1000)
    for l in range(NLAYERS):
        q, kv = _qkv(h, Wq[l], bq[l], Wk[l], bk[l], Wv[l], bv[l])
        e = _mm(edge_attr, We[l], be[l], 4000)
        qd, kvs = _sc_gather(q, kv, src3, dst3)
        msgden = _edge_math(qd, kvs, e)
        acc = _sc_scatter(dst2, msgden)
        h = _node_update(acc, h, Wskip[l], bskip[l],
                         bn_gamma[l], bn_beta[l])
    return _pool_mlp(h, batch3, reg_W1, reg_b1, reg_W2, reg_b2)


# speedup vs baseline: 24.5987x; 1.1579x over previous
import functools

import jax
import jax.numpy as jnp
from jax import lax
from jax.experimental import pallas as pl
from jax.experimental.pallas import tpu as pltpu
from jax.experimental.pallas import tpu_sc as plsc

N = 10000
E = 320000
DIN = 128
DE = 16
HID = 64
HEADS = 4
CH = 16
NLAYERS = 3
G = 16

NC = 2
NS = 16
NW = NC * NS
EPW = E // NW
CHUNK = 80
NCHUNK = EPW // CHUNK
AW = 128
HALF = N // NC
HROWS = 5120
RPT = HROWS // NS
TRASH = HALF + 16
CPS = E // CHUNK // NS

_mesh = plsc.VectorSubcoreMesh(core_axis_name="c", subcore_axis_name="s")


def _mm(x, w, b, bm):
    m, k = x.shape
    nout = w.shape[1]

    def body(xr, wr, br, outr):
        outr[...] = jnp.dot(xr[...], wr[...],
                            preferred_element_type=jnp.float32, precision=lax.Precision.HIGHEST) + br[...]

    return pl.pallas_call(
        body,
        grid=(m // bm,),
        in_specs=[
            pl.BlockSpec((bm, k), lambda i: (i, 0)),
            pl.BlockSpec((k, nout), lambda i: (0, 0)),
            pl.BlockSpec((1, nout), lambda i: (0, 0)),
        ],
        out_specs=pl.BlockSpec((bm, nout), lambda i: (i, 0)),
        out_shape=jax.ShapeDtypeStruct((m, nout), jnp.float32),
    )(x, w, b.reshape(1, nout))


def _qkv(h, wq, bq, wk, bk, wv, bv):
    bm = 1000

    def body(hr, wqr, bqr, wkr, bkr, wvr, bvr, qr, kvr):
        hh = hr[...]
        qq = jnp.dot(hh, wqr[...], preferred_element_type=jnp.float32, precision=lax.Precision.HIGHEST) + bqr[...]
        qr[...] = jnp.concatenate([qq, jnp.zeros((bm, HID), jnp.float32)], axis=1)
        kk = jnp.dot(hh, wkr[...], preferred_element_type=jnp.float32, precision=lax.Precision.HIGHEST) + bkr[...]
        vv = jnp.dot(hh, wvr[...], preferred_element_type=jnp.float32, precision=lax.Precision.HIGHEST) + bvr[...]
        kvr[...] = jnp.concatenate([kk, vv], axis=1)

    wspec = pl.BlockSpec((HID, HID), lambda i: (0, 0))
    bspec = pl.BlockSpec((1, HID), lambda i: (0, 0))
    return pl.pallas_call(
        body,
        grid=(N // bm,),
        in_specs=[pl.BlockSpec((bm, HID), lambda i: (i, 0)),
                  wspec, bspec, wspec, bspec, wspec, bspec],
        out_specs=[pl.BlockSpec((bm, 2 * HID), lambda i: (i, 0)),
                   pl.BlockSpec((bm, 2 * HID), lambda i: (i, 0))],
        out_shape=[jax.ShapeDtypeStruct((N, 2 * HID), jnp.float32),
                   jax.ShapeDtypeStruct((N, 2 * HID), jnp.float32)],
    )(h, wq, bq.reshape(1, HID), wk, bk.reshape(1, HID), wv, bv.reshape(1, HID))


def _sel():
    row = lax.broadcasted_iota(jnp.int32, (HID, HEADS), 0) // CH
    col = lax.broadcasted_iota(jnp.int32, (HID, HEADS), 1)
    return (row == col).astype(jnp.float32)


def _selT():
    row = lax.broadcasted_iota(jnp.int32, (HEADS, HID), 0)
    col = lax.broadcasted_iota(jnp.int32, (HEADS, HID), 1) // CH
    return (row == col).astype(jnp.float32)


def _edge_math(qd, kvs, e):
    be_blk = 4000

    def body(qdr, kvsr, er, outr):
        ks = kvsr[:, :HID]
        vs = kvsr[:, HID:]
        ee = er[...]
        p = qdr[:, :HID] * (ks + ee)
        alpha = jnp.dot(p, _sel(), preferred_element_type=jnp.float32, precision=lax.Precision.HIGHEST) * 0.25
        ex = jnp.exp(alpha)
        exb = jnp.dot(ex, _selT(), preferred_element_type=jnp.float32, precision=lax.Precision.HIGHEST)
        msg = (vs + ee) * exb
        outr[...] = jnp.concatenate(
            [msg, ex, jnp.zeros((be_blk, AW - HID - HEADS), jnp.float32)], axis=1)

    return pl.pallas_call(
        body,
        grid=(E // be_blk,),
        in_specs=[pl.BlockSpec((be_blk, 2 * HID), lambda i: (i, 0)),
                  pl.BlockSpec((be_blk, 2 * HID), lambda i: (i, 0)),
                  pl.BlockSpec((be_blk, HID), lambda i: (i, 0))],
        out_specs=pl.BlockSpec((be_blk, AW), lambda i: (i, 0)),
        out_shape=jax.ShapeDtypeStruct((E, AW), jnp.float32),
    )(qd, kvs, e)


def _node_update(acc, h, wskip, bskip, gamma, beta):
    bm = 1000
    bpc = HALF // bm

    def body(ar, hr, wr, br, gr, btr, outr):
        a = ar[...].reshape(bm, AW)
        num = a[:, :HID]
        den = a[:, HID:HID + HEADS]
        denb = jnp.dot(den, _selT(), preferred_element_type=jnp.float32, precision=lax.Precision.HIGHEST) + 1e-16
        hh = hr[...]
        out = num / denb + jnp.dot(hh, wr[...],
                                   preferred_element_type=jnp.float32, precision=lax.Precision.HIGHEST) + br[...]
        out = out * gr[...] + btr[...]
        g = out * 0.5 * (1.0 + lax.erf(out * 0.7071067811865476))
        outr[...] = g + hh

    return pl.pallas_call(
        body,
        grid=(N // bm,),
        in_specs=[pl.BlockSpec((1, bm, AW), lambda i: (i // bpc, i % bpc, 0)),
                  pl.BlockSpec((bm, HID), lambda i: (i, 0)),
                  pl.BlockSpec((HID, HID), lambda i: (0, 0)),
                  pl.BlockSpec((1, HID), lambda i: (0, 0)),
                  pl.BlockSpec((1, HID), lambda i: (0, 0)),
                  pl.BlockSpec((1, HID), lambda i: (0, 0))],
        out_specs=pl.BlockSpec((bm, HID), lambda i: (i, 0)),
        out_shape=jax.ShapeDtypeStruct((N, HID), jnp.float32),
    )(acc, h, wskip, bskip.reshape(1, HID),
      gamma.reshape(1, HID), beta.reshape(1, HID))


def _pool_mlp(h, batch3, w1, b1, w2, b2):
    bm = 1000
    steps = N // bm

    def body(hr, br, w1r, b1r, w2r, b2r, outr, sums, counts):
        i = pl.program_id(0)

        @pl.when(i == 0)
        def _init():
            sums[...] = jnp.zeros((G, HID), jnp.float32)
            counts[...] = jnp.zeros((G, 1), jnp.float32)

        bvals = br[...].reshape(1, bm)
        gids = lax.broadcasted_iota(jnp.int32, (G, bm), 0)
        onehot = (jnp.broadcast_to(bvals, (G, bm)) == gids).astype(jnp.float32)
        sums[...] += jnp.dot(onehot, hr[...], preferred_element_type=jnp.float32, precision=lax.Precision.HIGHEST)
        counts[...] += jnp.sum(onehot, axis=1, keepdims=True)

        @pl.when(i == steps - 1)
        def _fin():
            s = sums[...]
            mean = s / jnp.maximum(counts[...], 1.0)
            g = jnp.concatenate([mean, s], axis=1)
            r = jnp.maximum(
                jnp.dot(g, w1r[...], preferred_element_type=jnp.float32, precision=lax.Precision.HIGHEST) + b1r[...],
                0.0)
            outr[...] = jnp.dot(r, w2r[...],
                                preferred_element_type=jnp.float32, precision=lax.Precision.HIGHEST) + b2r[...]

    return pl.pallas_call(
        body,
        grid=(steps,),
        in_specs=[pl.BlockSpec((bm, HID), lambda i: (i, 0)),
                  pl.BlockSpec((1, 1, bm), lambda i: (i, 0, 0)),
                  pl.BlockSpec((2 * HID, HID), lambda i: (0, 0)),
                  pl.BlockSpec((1, HID), lambda i: (0, 0)),
                  pl.BlockSpec((HID, 1), lambda i: (0, 0)),
                  pl.BlockSpec((1, 1), lambda i: (0, 0))],
        out_specs=pl.BlockSpec((G, 1), lambda i: (0, 0)),
        out_shape=jax.ShapeDtypeStruct((G, 1), jnp.float32),
        scratch_shapes=[pltpu.VMEM((G, HID), jnp.float32),
                        pltpu.VMEM((G, 1), jnp.float32)],
    )(h, batch3, w1, b1.reshape(1, HID), w2, b2.reshape(1, 1))


@functools.partial(
    pl.kernel,
    out_type=[jax.ShapeDtypeStruct((E, 2 * HID), jnp.float32),
              jax.ShapeDtypeStruct((E, 2 * HID), jnp.float32)],
    mesh=_mesh,
    scratch_types=[pltpu.VMEM((NCHUNK, CHUNK), jnp.int32),
                   pltpu.VMEM((NCHUNK, CHUNK), jnp.int32),
                   pltpu.VMEM((CHUNK, 2 * HID), jnp.float32),
                   pltpu.VMEM((CHUNK, 2 * HID), jnp.float32),
                   pltpu.SemaphoreType.DMA,
                   pltpu.SemaphoreType.DMA],
)
def _sc_gather(q_hbm, kv_hbm, src3, dst3, qd_out, kvs_out,
               sidx, didx, qbuf, kvbuf, sem1, sem2):
    wid = lax.axis_index("s") * NC + lax.axis_index("c")
    pltpu.sync_copy(src3.at[wid], sidx)
    pltpu.sync_copy(dst3.at[wid], didx)

    def step(i, carry):
        base = wid * EPW + i * CHUNK
        d1 = pltpu.async_copy(q_hbm.at[didx.at[i]], qbuf, sem1)
        d2 = pltpu.async_copy(kv_hbm.at[sidx.at[i]], kvbuf, sem2)
        d1.wait()
        d2.wait()
        pltpu.sync_copy(qbuf, qd_out.at[pl.ds(base, CHUNK)])
        pltpu.sync_copy(kvbuf, kvs_out.at[pl.ds(base, CHUNK)])
        return carry

    lax.fori_loop(0, NCHUNK, step, 0)


@functools.partial(
    pl.kernel,
    out_type=jax.ShapeDtypeStruct((NC, HROWS, AW), jnp.float32),
    mesh=_mesh,
    scratch_types=[pltpu.VMEM((CPS, CHUNK), jnp.int32),
                   pltpu.VMEM((CPS, CHUNK), jnp.int32),
                   pltpu.VMEM((CHUNK, AW), jnp.float32),
                   pltpu.VMEM((40, AW), jnp.float32),
                   pltpu.VMEM_SHARED((HROWS, AW), jnp.float32)],
)
def _sc_scatter(dst2, msgden, out_hbm, idxbuf, lidxbuf, rowbuf, zbuf, shared):
    c = lax.axis_index("c")
    s = lax.axis_index("s")
    lo = c * HALF

    def zrow(i, carry):
        for j in range(AW // 16):
            zbuf[i, pl.ds(j * 16, 16)] = jnp.zeros((16,), jnp.float32)
        return carry

    lax.fori_loop(0, 40, zrow, 0)

    def zcopy(r, carry):
        pltpu.sync_copy(zbuf, shared.at[pl.ds(s * RPT + r * 40, 40)])
        return carry

    lax.fori_loop(0, RPT // 40, zcopy, 0)
    pltpu.sync_copy(dst2.at[s], idxbuf)

    def clamp(i, carry):
        for j in range(CHUNK // 16):
            v = idxbuf[i, pl.ds(j * 16, 16)] - lo
            ok = (v >= 0) & (v < HALF)
            lidxbuf[i, pl.ds(j * 16, 16)] = jnp.where(ok, v, TRASH)
        return carry

    lax.fori_loop(0, CPS, clamp, 0)
    plsc.subcore_barrier()

    def step(i, carry):
        ci = s * CPS + i
        pltpu.sync_copy(msgden.at[pl.ds(ci * CHUNK, CHUNK)], rowbuf)
        pltpu.sync_copy(rowbuf, shared.at[lidxbuf.at[i]], add=True)
        return carry

    lax.fori_loop(0, CPS, step, 0)
    plsc.subcore_barrier()
    pltpu.sync_copy(shared.at[pl.ds(s * RPT, RPT)],
                    out_hbm.at[c, pl.ds(s * RPT, RPT)])


def kernel(x, edge_index, edge_attr, batch, node_W, node_b, Wq, bq, Wk, bk,
           Wv, bv, We, be, Wskip, bskip, bn_gamma, bn_beta, reg_W1, reg_b1,
           reg_W2, reg_b2):
    src3 = edge_index[0].reshape(NW, NCHUNK, CHUNK)
    dst3 = edge_index[1].reshape(NW, NCHUNK, CHUNK)
    dst2 = edge_index[1].reshape(NS, CPS, CHUNK)
    batch3 = batch.reshape(N // 1000, 1, 1000)

    h = _mm(x, node_W, node_b, 1000)
    for l in range(NLAYERS):
        q, kv = _qkv(h, Wq[l], bq[l], Wk[l], bk[l], Wv[l], bv[l])
        e = _mm(edge_attr, We[l], be[l], 4000)
        qd, kvs = _sc_gather(q, kv, src3, dst3)
        msgden = _edge_math(qd, kvs, e)
        acc = _sc_scatter(dst2, msgden)
        h = _node_update(acc, h, Wskip[l], bskip[l],
                         bn_gamma[l], bn_beta[l])
    return _pool_mlp(h, batch3, reg_W1, reg_b1, reg_W2, reg_b2)


# speedup vs baseline: 27.6019x; 1.1221x over previous
import functools

import jax
import jax.numpy as jnp
from jax import lax
from jax.experimental import pallas as pl
from jax.experimental.pallas import tpu as pltpu
from jax.experimental.pallas import tpu_sc as plsc

N = 10000
E = 320000
DIN = 128
DE = 16
HID = 64
HEADS = 4
CH = 16
NLAYERS = 3
G = 16

NC = 2
NS = 16
NW = NC * NS
EPW = E // NW
CHUNK = 80
NCHUNK = EPW // CHUNK
AW = 128
HALF = N // NC
HROWS = 5120
RPT = HROWS // NS
TRASH = HALF + 16
CPS = E // CHUNK // NS

_mesh = plsc.VectorSubcoreMesh(core_axis_name="c", subcore_axis_name="s")


def _mm(x, w, b, bm):
    m, k = x.shape
    nout = w.shape[1]

    def body(xr, wr, br, outr):
        outr[...] = jnp.dot(xr[...], wr[...],
                            preferred_element_type=jnp.float32, precision=lax.Precision.HIGHEST) + br[...]

    return pl.pallas_call(
        body,
        grid=(m // bm,),
        in_specs=[
            pl.BlockSpec((bm, k), lambda i: (i, 0)),
            pl.BlockSpec((k, nout), lambda i: (0, 0)),
            pl.BlockSpec((1, nout), lambda i: (0, 0)),
        ],
        out_specs=pl.BlockSpec((bm, nout), lambda i: (i, 0)),
        out_shape=jax.ShapeDtypeStruct((m, nout), jnp.float32),
    )(x, w, b.reshape(1, nout))


def _qkv(h, wq, bq, wk, bk, wv, bv):
    bm = 1000

    def body(hr, wqr, bqr, wkr, bkr, wvr, bvr, qr, kvr):
        hh = hr[...]
        qq = jnp.dot(hh, wqr[...], preferred_element_type=jnp.float32, precision=lax.Precision.HIGHEST) + bqr[...]
        qr[...] = jnp.concatenate([qq, jnp.zeros((bm, HID), jnp.float32)], axis=1)
        kk = jnp.dot(hh, wkr[...], preferred_element_type=jnp.float32, precision=lax.Precision.HIGHEST) + bkr[...]
        vv = jnp.dot(hh, wvr[...], preferred_element_type=jnp.float32, precision=lax.Precision.HIGHEST) + bvr[...]
        kvr[...] = jnp.concatenate([kk, vv], axis=1)

    wspec = pl.BlockSpec((HID, HID), lambda i: (0, 0))
    bspec = pl.BlockSpec((1, HID), lambda i: (0, 0))
    return pl.pallas_call(
        body,
        grid=(N // bm,),
        in_specs=[pl.BlockSpec((bm, HID), lambda i: (i, 0)),
                  wspec, bspec, wspec, bspec, wspec, bspec],
        out_specs=[pl.BlockSpec((bm, 2 * HID), lambda i: (i, 0)),
                   pl.BlockSpec((bm, 2 * HID), lambda i: (i, 0))],
        out_shape=[jax.ShapeDtypeStruct((N, 2 * HID), jnp.float32),
                   jax.ShapeDtypeStruct((N, 2 * HID), jnp.float32)],
    )(h, wq, bq.reshape(1, HID), wk, bk.reshape(1, HID), wv, bv.reshape(1, HID))


def _sel():
    row = lax.broadcasted_iota(jnp.int32, (HID, HEADS), 0) // CH
    col = lax.broadcasted_iota(jnp.int32, (HID, HEADS), 1)
    return (row == col).astype(jnp.float32)


def _selT():
    row = lax.broadcasted_iota(jnp.int32, (HEADS, HID), 0)
    col = lax.broadcasted_iota(jnp.int32, (HEADS, HID), 1) // CH
    return (row == col).astype(jnp.float32)


def _edge_math(qd, kvs, e):
    be_blk = 4000

    def body(qdr, kvsr, er, outr):
        ks = kvsr[:, :HID]
        vs = kvsr[:, HID:]
        ee = er[...]
        p = qdr[:, :HID] * (ks + ee)
        alpha = jnp.dot(p, _sel(), preferred_element_type=jnp.float32, precision=lax.Precision.HIGHEST) * 0.25
        ex = jnp.exp(alpha)
        exb = jnp.dot(ex, _selT(), preferred_element_type=jnp.float32, precision=lax.Precision.HIGHEST)
        msg = (vs + ee) * exb
        outr[...] = jnp.concatenate(
            [msg, ex, jnp.zeros((be_blk, AW - HID - HEADS), jnp.float32)], axis=1)

    return pl.pallas_call(
        body,
        grid=(E // be_blk,),
        in_specs=[pl.BlockSpec((be_blk, 2 * HID), lambda i: (i, 0)),
                  pl.BlockSpec((be_blk, 2 * HID), lambda i: (i, 0)),
                  pl.BlockSpec((be_blk, HID), lambda i: (i, 0))],
        out_specs=pl.BlockSpec((be_blk, AW), lambda i: (i, 0)),
        out_shape=jax.ShapeDtypeStruct((E, AW), jnp.float32),
    )(qd, kvs, e)


def _node_update(acc, h, wskip, bskip, gamma, beta):
    bm = 1000
    bpc = HALF // bm

    def body(ar, hr, wr, br, gr, btr, outr):
        a = ar[...].reshape(bm, AW)
        num = a[:, :HID]
        den = a[:, HID:HID + HEADS]
        denb = jnp.dot(den, _selT(), preferred_element_type=jnp.float32, precision=lax.Precision.HIGHEST) + 1e-16
        hh = hr[...]
        out = num / denb + jnp.dot(hh, wr[...],
                                   preferred_element_type=jnp.float32, precision=lax.Precision.HIGHEST) + br[...]
        out = out * gr[...] + btr[...]
        g = out * 0.5 * (1.0 + lax.erf(out * 0.7071067811865476))
        outr[...] = g + hh

    return pl.pallas_call(
        body,
        grid=(N // bm,),
        in_specs=[pl.BlockSpec((1, bm, AW), lambda i: (i // bpc, i % bpc, 0)),
                  pl.BlockSpec((bm, HID), lambda i: (i, 0)),
                  pl.BlockSpec((HID, HID), lambda i: (0, 0)),
                  pl.BlockSpec((1, HID), lambda i: (0, 0)),
                  pl.BlockSpec((1, HID), lambda i: (0, 0)),
                  pl.BlockSpec((1, HID), lambda i: (0, 0))],
        out_specs=pl.BlockSpec((bm, HID), lambda i: (i, 0)),
        out_shape=jax.ShapeDtypeStruct((N, HID), jnp.float32),
    )(acc, h, wskip, bskip.reshape(1, HID),
      gamma.reshape(1, HID), beta.reshape(1, HID))


def _pool_mlp(h, batch3, w1, b1, w2, b2):
    bm = 1000
    steps = N // bm

    def body(hr, br, w1r, b1r, w2r, b2r, outr, sums, counts):
        i = pl.program_id(0)

        @pl.when(i == 0)
        def _init():
            sums[...] = jnp.zeros((G, HID), jnp.float32)
            counts[...] = jnp.zeros((G, 1), jnp.float32)

        bvals = br[...].reshape(1, bm)
        gids = lax.broadcasted_iota(jnp.int32, (G, bm), 0)
        onehot = (jnp.broadcast_to(bvals, (G, bm)) == gids).astype(jnp.float32)
        sums[...] += jnp.dot(onehot, hr[...], preferred_element_type=jnp.float32, precision=lax.Precision.HIGHEST)
        counts[...] += jnp.sum(onehot, axis=1, keepdims=True)

        @pl.when(i == steps - 1)
        def _fin():
            s = sums[...]
            mean = s / jnp.maximum(counts[...], 1.0)
            g = jnp.concatenate([mean, s], axis=1)
            r = jnp.maximum(
                jnp.dot(g, w1r[...], preferred_element_type=jnp.float32, precision=lax.Precision.HIGHEST) + b1r[...],
                0.0)
            outr[...] = jnp.dot(r, w2r[...],
                                preferred_element_type=jnp.float32, precision=lax.Precision.HIGHEST) + b2r[...]

    return pl.pallas_call(
        body,
        grid=(steps,),
        in_specs=[pl.BlockSpec((bm, HID), lambda i: (i, 0)),
                  pl.BlockSpec((1, 1, bm), lambda i: (i, 0, 0)),
                  pl.BlockSpec((2 * HID, HID), lambda i: (0, 0)),
                  pl.BlockSpec((1, HID), lambda i: (0, 0)),
                  pl.BlockSpec((HID, 1), lambda i: (0, 0)),
                  pl.BlockSpec((1, 1), lambda i: (0, 0))],
        out_specs=pl.BlockSpec((G, 1), lambda i: (0, 0)),
        out_shape=jax.ShapeDtypeStruct((G, 1), jnp.float32),
        scratch_shapes=[pltpu.VMEM((G, HID), jnp.float32),
                        pltpu.VMEM((G, 1), jnp.float32)],
    )(h, batch3, w1, b1.reshape(1, HID), w2, b2.reshape(1, 1))


@functools.partial(
    pl.kernel,
    out_type=[jax.ShapeDtypeStruct((E, 2 * HID), jnp.float32),
              jax.ShapeDtypeStruct((E, 2 * HID), jnp.float32)],
    mesh=_mesh,
    scratch_types=[pltpu.VMEM((NCHUNK, CHUNK), jnp.int32),
                   pltpu.VMEM((NCHUNK, CHUNK), jnp.int32),
                   pltpu.VMEM((2, CHUNK, 2 * HID), jnp.float32),
                   pltpu.VMEM((2, CHUNK, 2 * HID), jnp.float32),
                   pltpu.SemaphoreType.DMA,
                   pltpu.SemaphoreType.DMA,
                   pltpu.SemaphoreType.DMA,
                   pltpu.SemaphoreType.DMA,
                   pltpu.SemaphoreType.DMA,
                   pltpu.SemaphoreType.DMA,
                   pltpu.SemaphoreType.DMA,
                   pltpu.SemaphoreType.DMA],
)
def _sc_gather(q_hbm, kv_hbm, src3, dst3, qd_out, kvs_out,
               sidx, didx, qbuf, kvbuf,
               gq0, gq1, gkv0, gkv1, wq0, wq1, wkv0, wkv1):
    wid = lax.axis_index("s") * NC + lax.axis_index("c")
    gq = (gq0, gq1)
    gkv = (gkv0, gkv1)
    wq = (wq0, wq1)
    wkv = (wkv0, wkv1)
    pltpu.sync_copy(src3.at[wid], sidx)
    pltpu.sync_copy(dst3.at[wid], didx)

    def issue(ci, b):
        pltpu.async_copy(q_hbm.at[didx.at[ci]], qbuf.at[b], gq[b])
        pltpu.async_copy(kv_hbm.at[sidx.at[ci]], kvbuf.at[b], gkv[b])

    def wait_g(b):
        pltpu.make_async_copy(q_hbm.at[pl.ds(0, CHUNK)], qbuf.at[b], gq[b]).wait()
        pltpu.make_async_copy(kv_hbm.at[pl.ds(0, CHUNK)], kvbuf.at[b], gkv[b]).wait()

    def wr(ci, b):
        base = wid * EPW + ci * CHUNK
        pltpu.async_copy(qbuf.at[b], qd_out.at[pl.ds(base, CHUNK)], wq[b])
        pltpu.async_copy(kvbuf.at[b], kvs_out.at[pl.ds(base, CHUNK)], wkv[b])

    def wait_w(b):
        pltpu.make_async_copy(q_hbm.at[pl.ds(0, CHUNK)], qbuf.at[b], wq[b]).wait()
        pltpu.make_async_copy(kv_hbm.at[pl.ds(0, CHUNK)], kvbuf.at[b], wkv[b]).wait()

    issue(0, 0)
    issue(1, 1)

    def step(g, carry):
        c0 = 2 * g
        wait_g(0)
        wr(c0, 0)
        wait_w(0)
        issue(c0 + 2, 0)
        wait_g(1)
        wr(c0 + 1, 1)
        wait_w(1)
        issue(c0 + 3, 1)
        return carry

    lax.fori_loop(0, (NCHUNK - 3) // 2, step, 0)
    wait_g(0)
    wr(NCHUNK - 3, 0)
    wait_w(0)
    issue(NCHUNK - 1, 0)
    wait_g(1)
    wr(NCHUNK - 2, 1)
    wait_w(1)
    wait_g(0)
    wr(NCHUNK - 1, 0)
    wait_w(0)


@functools.partial(
    pl.kernel,
    out_type=jax.ShapeDtypeStruct((NC, HROWS, AW), jnp.float32),
    mesh=_mesh,
    scratch_types=[pltpu.VMEM((CPS, CHUNK), jnp.int32),
                   pltpu.VMEM((CPS, CHUNK), jnp.int32),
                   pltpu.VMEM((2, CHUNK, AW), jnp.float32),
                   pltpu.VMEM((8, AW), jnp.float32),
                   pltpu.VMEM_SHARED((HROWS, AW), jnp.float32),
                   pltpu.SemaphoreType.DMA,
                   pltpu.SemaphoreType.DMA,
                   pltpu.SemaphoreType.DMA,
                   pltpu.SemaphoreType.DMA],
)
def _sc_scatter(dst2, msgden, out_hbm, idxbuf, lidxbuf, rowbuf, zbuf, shared,
                lr0, lr1, sa0, sa1):
    c = lax.axis_index("c")
    s = lax.axis_index("s")
    lo = c * HALF
    lr = (lr0, lr1)
    sa = (sa0, sa1)

    def zrow(i, carry):
        for j in range(AW // 16):
            zbuf[i, pl.ds(j * 16, 16)] = jnp.zeros((16,), jnp.float32)
        return carry

    lax.fori_loop(0, 8, zrow, 0)

    def zcopy(r, carry):
        pltpu.sync_copy(zbuf, shared.at[pl.ds(s * RPT + r * 8, 8)])
        return carry

    lax.fori_loop(0, RPT // 8, zcopy, 0)
    pltpu.sync_copy(dst2.at[s], idxbuf)

    def clamp(i, carry):
        for j in range(CHUNK // 16):
            v = idxbuf[i, pl.ds(j * 16, 16)] - lo
            ok = (v >= 0) & (v < HALF)
            lidxbuf[i, pl.ds(j * 16, 16)] = jnp.where(ok, v, TRASH)
        return carry

    lax.fori_loop(0, CPS, clamp, 0)
    plsc.subcore_barrier()

    def load(i, b):
        pltpu.async_copy(msgden.at[pl.ds((s * CPS + i) * CHUNK, CHUNK)],
                         rowbuf.at[b], lr[b])

    def wait_l(b):
        pltpu.make_async_copy(msgden.at[pl.ds(0, CHUNK)], rowbuf.at[b],
                              lr[b]).wait()

    def add(i, b):
        pltpu.async_copy(rowbuf.at[b], shared.at[lidxbuf.at[i]], sa[b],
                         add=True)

    def wait_a(b):
        pltpu.make_async_copy(msgden.at[pl.ds(0, CHUNK)], rowbuf.at[b],
                              sa[b]).wait()

    load(0, 0)
    load(1, 1)

    def step(g, carry):
        c0 = 2 * g
        wait_l(0)
        add(c0, 0)
        wait_a(0)
        load(c0 + 2, 0)
        wait_l(1)
        add(c0 + 1, 1)
        wait_a(1)
        load(c0 + 3, 1)
        return carry

    lax.fori_loop(0, (CPS - 2) // 2, step, 0)
    wait_l(0)
    add(CPS - 2, 0)
    wait_a(0)
    wait_l(1)
    add(CPS - 1, 1)
    wait_a(1)
    plsc.subcore_barrier()
    pltpu.sync_copy(shared.at[pl.ds(s * RPT, RPT)],
                    out_hbm.at[c, pl.ds(s * RPT, RPT)])


def kernel(x, edge_index, edge_attr, batch, node_W, node_b, Wq, bq, Wk, bk,
           Wv, bv, We, be, Wskip, bskip, bn_gamma, bn_beta, reg_W1, reg_b1,
           reg_W2, reg_b2):
    src3 = edge_index[0].reshape(NW, NCHUNK, CHUNK)
    dst3 = edge_index[1].reshape(NW, NCHUNK, CHUNK)
    dst2 = edge_index[1].reshape(NS, CPS, CHUNK)
    batch3 = batch.reshape(N // 1000, 1, 1000)

    h = _mm(x, node_W, node_b, 1000)
    for l in range(NLAYERS):
        q, kv = _qkv(h, Wq[l], bq[l], Wk[l], bk[l], Wv[l], bv[l])
        e = _mm(edge_attr, We[l], be[l], 4000)
        qd, kvs = _sc_gather(q, kv, src3, dst3)
        msgden = _edge_math(qd, kvs, e)
        acc = _sc_scatter(dst2, msgden)
        h = _node_update(acc, h, Wskip[l], bskip[l],
                         bn_gamma[l], bn_beta[l])
    return _pool_mlp(h, batch3, reg_W1, reg_b1, reg_W2, reg_b2)
